# baseline (speedup 1.0000x reference)
import jax
import jax.numpy as jnp
from jax.experimental import pallas as pl

_BN = 400


def _body(src_ref, na_ref, nb_ref, wa_ref, ws_ref, out_ref):
    s = jnp.sum(na_ref[...], axis=1) + jnp.sum(nb_ref[...], axis=1)
    deg = na_ref.shape[1] + nb_ref.shape[1]
    mean = s * (1.0 / deg)
    h = jnp.dot(mean, wa_ref[...], preferred_element_type=jnp.float32)
    h += jnp.dot(src_ref[...], ws_ref[...], preferred_element_type=jnp.float32)
    out_ref[...] = jnp.maximum(h, 0.0)


def kernel(src_node_features, neighbor_node_features, W_agg, W_self):
    n, deg, d_in = neighbor_node_features.shape
    d_hid = W_agg.shape[1]
    hd = deg // 2
    grid = (n // _BN,)
    return pl.pallas_call(
        _body,
        grid=grid,
        in_specs=[
            pl.BlockSpec((_BN, d_in), lambda i: (i, 0)),
            pl.BlockSpec((_BN, hd, d_in), lambda i: (i, 0, 0)),
            pl.BlockSpec((_BN, hd, d_in), lambda i: (i, 1, 0)),
            pl.BlockSpec((d_in, d_hid), lambda i: (0, 0)),
            pl.BlockSpec((d_in, d_hid), lambda i: (0, 0)),
        ],
        out_specs=pl.BlockSpec((_BN, d_hid), lambda i: (i, 0)),
        out_shape=jax.ShapeDtypeStruct((n, d_hid), jnp.float32),
    )(src_node_features, neighbor_node_features, neighbor_node_features,
      W_agg, W_self)
